# Initial kernel scaffold; baseline (speedup 1.0000x reference)
import jax
import jax.numpy as jnp
from jax.experimental import pallas as pl


def kernel(h, edge_index, e, W_embed, b_embed, W0, b0, W1, b1, W2, b2, p_pos, p_neg, FC_w):
    raise NotImplementedError("write your pallas kernel here")


# speedup vs baseline: 4.7434x; 4.7434x over previous
import functools

import jax
import jax.numpy as jnp
from jax import lax
from jax.experimental import pallas as pl
from jax.experimental.pallas import tpu as pltpu
from jax.experimental.pallas import tpu_sc as plsc

_N = 10000
_E = 320000
_H = 128
_NC = 2
_NS = 16
_NW = _NC * _NS
_EW = _E // _NW
_CH = 80
_NCHUNK = _EW // _CH
_RT = 624
_REM0 = _NS * _RT
_REM = _N - _REM0

_BLK = 1000
_NBLK = _N // _BLK

_f32 = jnp.float32


def _sc_body(with_deg, *refs):
    if with_deg:
        (h, src, dst, zrows, zdeg, part, degpart,
         acc, dacc, idx, didx, rows, ones_v, degv, sem) = refs
    else:
        (h, src, dst, zrows, part,
         acc, idx, didx, rows, sem) = refs

    c = lax.axis_index("c")
    s = lax.axis_index("s")
    wid = s * _NC + c

    pltpu.sync_copy(zrows.at[pl.ds(s * _RT, _RT)], acc.at[pl.ds(s * _RT, _RT)])

    @pl.when(s == 0)
    def _():
        pltpu.sync_copy(zrows.at[pl.ds(_REM0, _REM)],
                        acc.at[pl.ds(_REM0, _REM)])
    if with_deg:
        @pl.when(s == 0)
        def _():
            pltpu.sync_copy(zdeg, degv)
            pltpu.sync_copy(degv, dacc)
        for k in range(_CH // 16):
            ones_v[pl.ds(k * 16, 16)] = jnp.ones((16,), _f32)
    plsc.subcore_barrier()

    base = wid * _EW

    def chunk(i, carry):
        off = base + i * _CH
        pltpu.sync_copy(src.at[pl.ds(off, _CH)], idx)
        pltpu.async_copy(h.at[idx], rows, sem).wait()
        pltpu.sync_copy(dst.at[pl.ds(off, _CH)], didx)
        pltpu.sync_copy(rows, acc.at[didx], add=True)
        if with_deg:
            pltpu.sync_copy(ones_v, dacc.at[didx], add=True)
        return carry

    lax.fori_loop(0, _NCHUNK, chunk, 0)
    plsc.subcore_barrier()

    pltpu.sync_copy(acc.at[pl.ds(s * _RT, _RT)],
                    part.at[pl.ds(c * _N + s * _RT, _RT)])

    @pl.when(s == 0)
    def _():
        pltpu.sync_copy(acc.at[pl.ds(_REM0, _REM)],
                        part.at[pl.ds(c * _N + _REM0, _REM)])
    if with_deg:
        @pl.when(s == 0)
        def _():
            pltpu.sync_copy(dacc, degv)
            pltpu.sync_copy(degv, degpart.at[pl.ds(c * _N, _N)])


def _make_sc(with_deg):
    mesh = plsc.VectorSubcoreMesh(core_axis_name="c", subcore_axis_name="s")
    out_type = [jax.ShapeDtypeStruct((2 * _N, _H), _f32)]
    scratch = [
        pltpu.VMEM_SHARED((_N, _H), _f32),
    ]
    if with_deg:
        out_type.append(jax.ShapeDtypeStruct((2 * _N,), _f32))
        scratch.append(pltpu.VMEM_SHARED((_N,), _f32))
    scratch += [
        pltpu.VMEM((_CH,), jnp.int32),
        pltpu.VMEM((_CH,), jnp.int32),
        pltpu.VMEM((_CH, _H), _f32),
    ]
    if with_deg:
        scratch.append(pltpu.VMEM((_CH,), _f32))
        scratch.append(pltpu.VMEM((_N,), _f32))
    scratch.append(pltpu.SemaphoreType.DMA)
    return pl.kernel(
        functools.partial(_sc_body, with_deg),
        out_type=out_type,
        mesh=mesh,
        scratch_types=scratch,
    )


def _embed_body(h_ref, wt_ref, b_ref, o_ref):
    o_ref[...] = (
        jnp.dot(h_ref[...], wt_ref[...], preferred_element_type=_f32)
        + b_ref[...]
    )


def _layer_body(first, last, *refs):
    if first:
        (h_ref, p0_ref, p1_ref, d0_ref, d1_ref, wl_ref, wr_ref, b_ref,
         o_ref, inv_ref) = refs
        inv = 1.0 / jnp.maximum(d0_ref[...] + d1_ref[...], 1.0)
    elif last:
        (h_ref, p0_ref, p1_ref, inv_in_ref, wl_ref, wr_ref, b_ref,
         o_ref, cs_ref) = refs
        inv = inv_in_ref[...]
    else:
        (h_ref, p0_ref, p1_ref, inv_in_ref, wl_ref, wr_ref, b_ref,
         o_ref) = refs
        inv = inv_in_ref[...]
    h = h_ref[...]
    agg = (p0_ref[...] + p1_ref[...]) * inv
    bundle = (
        jnp.dot(h, wl_ref[...], preferred_element_type=_f32)
        + jnp.dot(agg, wr_ref[...], preferred_element_type=_f32)
        + b_ref[...]
    )
    nrm = jnp.maximum(
        jnp.sqrt(jnp.sum(bundle * bundle, axis=1, keepdims=True)), 1e-12)
    out = h + jnp.maximum(bundle / nrm, 0.0)
    o_ref[...] = out
    if first:
        inv_ref[...] = inv
    if last:
        i = pl.program_id(0)
        colsum = jnp.sum(out, axis=0, keepdims=True)

        @pl.when(i == 0)
        def _():
            cs_ref[...] = colsum

        @pl.when(i > 0)
        def _():
            cs_ref[...] += colsum


def _readout_body(cs_ref, pp_ref, pn_ref, fc_ref, o_ref):
    hg = cs_ref[...] * (1.0 / _N)
    dp = jnp.sum((hg - pp_ref[...]) ** 2, axis=1, keepdims=True)
    dn = jnp.sum((hg - pn_ref[...]) ** 2, axis=1, keepdims=True)
    d = jnp.concatenate([dp, dn], axis=0)
    ss = jnp.log((d + 1.0) / (d + 1e-12))
    y = jnp.sum(ss * fc_ref[...])
    o_ref[...] = jnp.reshape(jax.nn.sigmoid(y), (1, 1))


def _row_spec(off=0):
    return pl.BlockSpec((_BLK, _H), lambda i: (i + off, 0))


def _full_spec(shape):
    return pl.BlockSpec(shape, lambda i: tuple(0 for _ in shape))


def _embed_call(h, wt, b):
    return pl.pallas_call(
        _embed_body,
        grid=(_NBLK,),
        in_specs=[_row_spec(), _full_spec((_H, _H)), _full_spec((1, _H))],
        out_specs=_row_spec(),
        out_shape=jax.ShapeDtypeStruct((_N, _H), _f32),
    )(h, wt, b)


def _layer_call(first, last, h, part, deg_or_inv, wl, wr, b):
    col_spec = pl.BlockSpec((_BLK, 1), lambda i: (i, 0))
    col2_spec0 = pl.BlockSpec((_BLK, 1), lambda i: (i, 0))
    col2_spec1 = pl.BlockSpec((_BLK, 1), lambda i: (i + _NBLK, 0))
    in_specs = [
        _row_spec(),
        _row_spec(),
        _row_spec(_NBLK),
    ]
    args = [h, part, part]
    if first:
        in_specs += [col2_spec0, col2_spec1]
        args += [deg_or_inv, deg_or_inv]
    else:
        in_specs += [col_spec]
        args += [deg_or_inv]
    in_specs += [_full_spec((_H, _H)), _full_spec((_H, _H)),
                 _full_spec((1, _H))]
    args += [wl, wr, b]
    out_specs = [_row_spec()]
    out_shape = [jax.ShapeDtypeStruct((_N, _H), _f32)]
    if first:
        out_specs.append(pl.BlockSpec((_BLK, 1), lambda i: (i, 0)))
        out_shape.append(jax.ShapeDtypeStruct((_N, 1), _f32))
    if last:
        out_specs.append(pl.BlockSpec((1, _H), lambda i: (0, 0)))
        out_shape.append(jax.ShapeDtypeStruct((1, _H), _f32))
    outs = pl.pallas_call(
        functools.partial(_layer_body, first, last),
        grid=(_NBLK,),
        in_specs=in_specs,
        out_specs=out_specs,
        out_shape=out_shape,
    )(*args)
    return outs


def _readout_call(cs, pp, pn, fc):
    return pl.pallas_call(
        _readout_body,
        out_shape=jax.ShapeDtypeStruct((1, 1), _f32),
    )(cs, pp, pn, fc)


_sc_seg_deg = _make_sc(True)
_sc_seg = _make_sc(False)


def kernel(h, edge_index, e, W_embed, b_embed, W0, b0, W1, b1, W2, b2,
           p_pos, p_neg, FC_w):
    src = edge_index[0]
    dst = edge_index[1]
    zrows = jnp.zeros((_N, _H), _f32)
    zdeg = jnp.zeros((_N,), _f32)

    h0 = _embed_call(h, W_embed.T, b_embed.reshape(1, _H))

    part1, degpart = _sc_seg_deg(h0, src, dst, zrows, zdeg)
    h1, inv = _layer_call(True, False, h0, part1,
                          degpart.reshape(2 * _N, 1),
                          W0[:, :_H].T, W0[:, _H:].T, b0.reshape(1, _H))

    part2, = _sc_seg(h1, src, dst, zrows)
    h2, = _layer_call(False, False, h1, part2, inv,
                      W1[:, :_H].T, W1[:, _H:].T, b1.reshape(1, _H))

    part3, = _sc_seg(h2, src, dst, zrows)
    _h3, cs = _layer_call(False, True, h2, part3, inv,
                          W2[:, :_H].T, W2[:, _H:].T, b2.reshape(1, _H))

    y = _readout_call(cs, p_pos, p_neg, FC_w.reshape(8, 1))
    return y.reshape(())


# speedup vs baseline: 11.4981x; 2.4240x over previous
import functools

import jax
import jax.numpy as jnp
from jax import lax
from jax.experimental import pallas as pl
from jax.experimental.pallas import tpu as pltpu
from jax.experimental.pallas import tpu_sc as plsc

_N = 10000
_E = 320000
_H = 128
_NC = 2
_NS = 16
_NW = _NC * _NS
_EW = _E // _NW
_CH = 128
_EWM = 9984
_NCHUNK = _EWM // _CH
_NDBL = _NCHUNK // 2
_XBASE = _NW * _EWM
_RT = 624
_REM0 = _NS * _RT
_REM = _N - _REM0

_BLK = 1000
_NBLK = _N // _BLK

_f32 = jnp.float32


def _sc_body(with_deg, *refs):
    if with_deg:
        (h, src, dst, zrows, zdeg, part, degpart,
         acc, dacc, idx0, idx1, didx0, didx1, didx2, didx3, rows0, rows1,
         ones_v, degv,
         isem0, isem1, dsem0, dsem1, dsem2, dsem3,
         gsem0, gsem1, ssem0, ssem1) = refs
    else:
        (h, src, dst, zrows, part,
         acc, idx0, idx1, didx0, didx1, didx2, didx3, rows0, rows1,
         isem0, isem1, dsem0, dsem1, dsem2, dsem3,
         gsem0, gsem1, ssem0, ssem1) = refs
    idx = (idx0, idx1)
    didx = (didx0, didx1, didx2, didx3)
    rows = (rows0, rows1)
    isem = (isem0, isem1)
    dsem = (dsem0, dsem1, dsem2, dsem3)
    gsem = (gsem0, gsem1)
    ssem = (ssem0, ssem1)

    c = lax.axis_index("c")
    s = lax.axis_index("s")
    wid = s * _NC + c

    pltpu.sync_copy(zrows.at[pl.ds(s * _RT, _RT)], acc.at[pl.ds(s * _RT, _RT)])

    @pl.when(s == 0)
    def _():
        pltpu.sync_copy(zrows.at[pl.ds(_REM0, _REM)],
                        acc.at[pl.ds(_REM0, _REM)])
    if with_deg:
        @pl.when(s == 0)
        def _():
            pltpu.sync_copy(zdeg, degv)
            pltpu.sync_copy(degv, dacc)
        for k in range(_CH // 16):
            ones_v[pl.ds(k * 16, 16)] = jnp.ones((16,), _f32)
    plsc.subcore_barrier()

    base = wid * _EWM

    def load_idx(i, b, d):
        off = base + i * _CH
        pltpu.async_copy(src.at[pl.ds(off, _CH)], idx[b], isem[b])
        pltpu.async_copy(dst.at[pl.ds(off, _CH)], didx[d], dsem[d])

    def wait_idx(b):
        pltpu.make_async_copy(src.at[pl.ds(0, _CH)], idx[b], isem[b]).wait()

    def wait_didx(d):
        pltpu.make_async_copy(dst.at[pl.ds(0, _CH)], didx[d], dsem[d]).wait()

    def start_gather(b):
        pltpu.async_copy(h.at[idx[b]], rows[b], gsem[b])

    def wait_gather(b):
        pltpu.make_async_copy(h.at[idx[b]], rows[b], gsem[b]).wait()

    def start_scatter(b, d):
        pltpu.async_copy(rows[b], acc.at[didx[d]], ssem[b], add=True)

    def wait_scatter(b, d):
        pltpu.make_async_copy(rows[b], acc.at[didx[d]], ssem[b]).wait()

    load_idx(0, 0, 0)
    load_idx(1, 1, 1)
    wait_idx(0)
    start_gather(0)

    def dbl(j, carry):
        for b in (0, 1):
            i = 2 * j + b

            @pl.when(i + 1 < _NCHUNK)
            def _():
                wait_idx(1 - b)

                @pl.when(i >= 1)
                def _():
                    wait_scatter(1 - b, 0)

                start_gather(1 - b)

            wait_gather(b)
            for par in (0, 1):
                dd = 2 * par + b

                @pl.when(j % 2 == par)
                def _():
                    wait_didx(dd)
                    start_scatter(b, dd)
                    if with_deg:
                        pltpu.sync_copy(ones_v, dacc.at[didx[dd]], add=True)

                    @pl.when(i + 2 < _NCHUNK)
                    def _():
                        load_idx(i + 2, b, (dd + 2) % 4)
        return carry

    lax.fori_loop(0, _NDBL, dbl, 0)
    wait_scatter(0, 0)
    wait_scatter(1, 1)

    @pl.when(wid < 4)
    def _():
        off = _XBASE + wid * _CH
        pltpu.sync_copy(src.at[pl.ds(off, _CH)], idx0)
        pltpu.async_copy(h.at[idx0], rows0, gsem0).wait()
        pltpu.sync_copy(dst.at[pl.ds(off, _CH)], didx0)
        pltpu.sync_copy(rows0, acc.at[didx0], add=True)
        if with_deg:
            pltpu.sync_copy(ones_v, dacc.at[didx0], add=True)

    plsc.subcore_barrier()

    pltpu.sync_copy(acc.at[pl.ds(s * _RT, _RT)],
                    part.at[pl.ds(c * _N + s * _RT, _RT)])

    @pl.when(s == 0)
    def _():
        pltpu.sync_copy(acc.at[pl.ds(_REM0, _REM)],
                        part.at[pl.ds(c * _N + _REM0, _REM)])
    if with_deg:
        @pl.when(s == 0)
        def _():
            pltpu.sync_copy(dacc, degv)
            pltpu.sync_copy(degv, degpart.at[pl.ds(c * _N, _N)])


def _make_sc(with_deg):
    mesh = plsc.VectorSubcoreMesh(core_axis_name="c", subcore_axis_name="s")
    out_type = [jax.ShapeDtypeStruct((2 * _N, _H), _f32)]
    scratch = [
        pltpu.VMEM_SHARED((_N, _H), _f32),
    ]
    if with_deg:
        out_type.append(jax.ShapeDtypeStruct((2 * _N,), _f32))
        scratch.append(pltpu.VMEM_SHARED((_N,), _f32))
    scratch += [
        pltpu.VMEM((_CH,), jnp.int32),
        pltpu.VMEM((_CH,), jnp.int32),
        pltpu.VMEM((_CH,), jnp.int32),
        pltpu.VMEM((_CH,), jnp.int32),
        pltpu.VMEM((_CH,), jnp.int32),
        pltpu.VMEM((_CH,), jnp.int32),
        pltpu.VMEM((_CH, _H), _f32),
        pltpu.VMEM((_CH, _H), _f32),
    ]
    if with_deg:
        scratch.append(pltpu.VMEM((_CH,), _f32))
        scratch.append(pltpu.VMEM((_N,), _f32))
    scratch += [pltpu.SemaphoreType.DMA] * 10
    return pl.kernel(
        functools.partial(_sc_body, with_deg),
        out_type=out_type,
        mesh=mesh,
        scratch_types=scratch,
    )


def _embed_body(h_ref, wt_ref, b_ref, o_ref):
    o_ref[...] = (
        jnp.dot(h_ref[...], wt_ref[...], preferred_element_type=_f32)
        + b_ref[...]
    )


def _layer_body(first, last, *refs):
    if first:
        (h_ref, p0_ref, p1_ref, d0_ref, d1_ref, wl_ref, wr_ref, b_ref,
         o_ref, inv_ref) = refs
        inv = 1.0 / jnp.maximum(d0_ref[...] + d1_ref[...], 1.0)
    elif last:
        (h_ref, p0_ref, p1_ref, inv_in_ref, wl_ref, wr_ref, b_ref,
         o_ref, cs_ref) = refs
        inv = inv_in_ref[...]
    else:
        (h_ref, p0_ref, p1_ref, inv_in_ref, wl_ref, wr_ref, b_ref,
         o_ref) = refs
        inv = inv_in_ref[...]
    h = h_ref[...]
    agg = (p0_ref[...] + p1_ref[...]) * inv
    bundle = (
        jnp.dot(h, wl_ref[...], preferred_element_type=_f32)
        + jnp.dot(agg, wr_ref[...], preferred_element_type=_f32)
        + b_ref[...]
    )
    nrm = jnp.maximum(
        jnp.sqrt(jnp.sum(bundle * bundle, axis=1, keepdims=True)), 1e-12)
    out = h + jnp.maximum(bundle / nrm, 0.0)
    o_ref[...] = out
    if first:
        inv_ref[...] = inv
    if last:
        i = pl.program_id(0)
        colsum = jnp.sum(out, axis=0, keepdims=True)

        @pl.when(i == 0)
        def _():
            cs_ref[...] = colsum

        @pl.when(i > 0)
        def _():
            cs_ref[...] += colsum


def _readout_body(cs_ref, pp_ref, pn_ref, fc_ref, o_ref):
    hg = cs_ref[...] * (1.0 / _N)
    dp = jnp.sum((hg - pp_ref[...]) ** 2, axis=1, keepdims=True)
    dn = jnp.sum((hg - pn_ref[...]) ** 2, axis=1, keepdims=True)
    d = jnp.concatenate([dp, dn], axis=0)
    ss = jnp.log((d + 1.0) / (d + 1e-12))
    y = jnp.sum(ss * fc_ref[...])
    o_ref[...] = jnp.reshape(jax.nn.sigmoid(y), (1, 1))


def _row_spec(off=0):
    return pl.BlockSpec((_BLK, _H), lambda i: (i + off, 0))


def _full_spec(shape):
    return pl.BlockSpec(shape, lambda i: tuple(0 for _ in shape))


def _embed_call(h, wt, b):
    return pl.pallas_call(
        _embed_body,
        grid=(_NBLK,),
        in_specs=[_row_spec(), _full_spec((_H, _H)), _full_spec((1, _H))],
        out_specs=_row_spec(),
        out_shape=jax.ShapeDtypeStruct((_N, _H), _f32),
    )(h, wt, b)


def _layer_call(first, last, h, part, deg_or_inv, wl, wr, b):
    col_spec = pl.BlockSpec((_BLK, 1), lambda i: (i, 0))
    col2_spec0 = pl.BlockSpec((_BLK, 1), lambda i: (i, 0))
    col2_spec1 = pl.BlockSpec((_BLK, 1), lambda i: (i + _NBLK, 0))
    in_specs = [
        _row_spec(),
        _row_spec(),
        _row_spec(_NBLK),
    ]
    args = [h, part, part]
    if first:
        in_specs += [col2_spec0, col2_spec1]
        args += [deg_or_inv, deg_or_inv]
    else:
        in_specs += [col_spec]
        args += [deg_or_inv]
    in_specs += [_full_spec((_H, _H)), _full_spec((_H, _H)),
                 _full_spec((1, _H))]
    args += [wl, wr, b]
    out_specs = [_row_spec()]
    out_shape = [jax.ShapeDtypeStruct((_N, _H), _f32)]
    if first:
        out_specs.append(pl.BlockSpec((_BLK, 1), lambda i: (i, 0)))
        out_shape.append(jax.ShapeDtypeStruct((_N, 1), _f32))
    if last:
        out_specs.append(pl.BlockSpec((1, _H), lambda i: (0, 0)))
        out_shape.append(jax.ShapeDtypeStruct((1, _H), _f32))
    outs = pl.pallas_call(
        functools.partial(_layer_body, first, last),
        grid=(_NBLK,),
        in_specs=in_specs,
        out_specs=out_specs,
        out_shape=out_shape,
    )(*args)
    return outs


def _readout_call(cs, pp, pn, fc):
    return pl.pallas_call(
        _readout_body,
        out_shape=jax.ShapeDtypeStruct((1, 1), _f32),
    )(cs, pp, pn, fc)


_sc_seg_deg = _make_sc(True)
_sc_seg = _make_sc(False)


def kernel(h, edge_index, e, W_embed, b_embed, W0, b0, W1, b1, W2, b2,
           p_pos, p_neg, FC_w):
    src = edge_index[0]
    dst = edge_index[1]
    zrows = jnp.zeros((_N, _H), _f32)
    zdeg = jnp.zeros((_N,), _f32)

    h0 = _embed_call(h, W_embed.T, b_embed.reshape(1, _H))

    part1, degpart = _sc_seg_deg(h0, src, dst, zrows, zdeg)
    h1, inv = _layer_call(True, False, h0, part1,
                          degpart.reshape(2 * _N, 1),
                          W0[:, :_H].T, W0[:, _H:].T, b0.reshape(1, _H))

    part2, = _sc_seg(h1, src, dst, zrows)
    h2, = _layer_call(False, False, h1, part2, inv,
                      W1[:, :_H].T, W1[:, _H:].T, b1.reshape(1, _H))

    part3, = _sc_seg(h2, src, dst, zrows)
    _h3, cs = _layer_call(False, True, h2, part3, inv,
                          W2[:, :_H].T, W2[:, _H:].T, b2.reshape(1, _H))

    y = _readout_call(cs, p_pos, p_neg, FC_w.reshape(8, 1))
    return y.reshape(())


# speedup vs baseline: 11.5914x; 1.0081x over previous
import functools

import jax
import jax.numpy as jnp
from jax import lax
from jax.experimental import pallas as pl
from jax.experimental.pallas import tpu as pltpu
from jax.experimental.pallas import tpu_sc as plsc

_N = 10000
_E = 320000
_H = 128
_NC = 2
_NS = 16
_NW = _NC * _NS
_EW = _E // _NW
_CH = 128
_EWM = 9984
_NCHUNK = _EWM // _CH
_NDBL = _NCHUNK // 2
_XBASE = _NW * _EWM
_RT = 624
_REM0 = _NS * _RT
_REM = _N - _REM0

_BLK = 1000
_NBLK = _N // _BLK

_f32 = jnp.float32


def _sc_body(with_deg, *refs):
    if with_deg:
        (h, src, dst, zrows, zdeg, part, degpart,
         acc, dacc, idx0, idx1, didx0, didx1, didx2, didx3, rows0, rows1,
         ones_v, degv,
         isem0, isem1, dsem0, dsem1, dsem2, dsem3,
         gsem0, gsem1, ssem0, ssem1) = refs
    else:
        (h, src, dst, zrows, part,
         acc, idx0, idx1, didx0, didx1, didx2, didx3, rows0, rows1,
         isem0, isem1, dsem0, dsem1, dsem2, dsem3,
         gsem0, gsem1, ssem0, ssem1) = refs
    idx = (idx0, idx1)
    didx = (didx0, didx1, didx2, didx3)
    rows = (rows0, rows1)
    isem = (isem0, isem1)
    dsem = (dsem0, dsem1, dsem2, dsem3)
    gsem = (gsem0, gsem1)
    ssem = (ssem0, ssem1)

    c = lax.axis_index("c")
    s = lax.axis_index("s")
    wid = s * _NC + c

    pltpu.sync_copy(zrows.at[pl.ds(s * _RT, _RT)], acc.at[pl.ds(s * _RT, _RT)])

    @pl.when(s == 0)
    def _():
        pltpu.sync_copy(zrows.at[pl.ds(_REM0, _REM)],
                        acc.at[pl.ds(_REM0, _REM)])
    if with_deg:
        @pl.when(s == 0)
        def _():
            pltpu.sync_copy(zdeg, degv)
            pltpu.sync_copy(degv, dacc)
        for k in range(_CH // 16):
            ones_v[pl.ds(k * 16, 16)] = jnp.ones((16,), _f32)
    plsc.subcore_barrier()

    base = wid * _EWM

    def load_idx(i, b, d):
        off = base + i * _CH
        pltpu.async_copy(src.at[pl.ds(off, _CH)], idx[b], isem[b])
        pltpu.async_copy(dst.at[pl.ds(off, _CH)], didx[d], dsem[d])

    def wait_idx(b):
        pltpu.make_async_copy(src.at[pl.ds(0, _CH)], idx[b], isem[b]).wait()

    def wait_didx(d):
        pltpu.make_async_copy(dst.at[pl.ds(0, _CH)], didx[d], dsem[d]).wait()

    def start_gather(b):
        pltpu.async_copy(h.at[idx[b]], rows[b], gsem[b])

    def wait_gather(b):
        pltpu.make_async_copy(h.at[idx[b]], rows[b], gsem[b]).wait()

    def start_scatter(b, d):
        pltpu.async_copy(rows[b], acc.at[didx[d]], ssem[b], add=True)

    def wait_scatter(b, d):
        pltpu.make_async_copy(rows[b], acc.at[didx[d]], ssem[b]).wait()

    load_idx(0, 0, 0)
    load_idx(1, 1, 1)
    wait_idx(0)
    start_gather(0)

    def dbl(j, carry):
        for b in (0, 1):
            i = 2 * j + b

            @pl.when(i + 1 < _NCHUNK)
            def _():
                wait_idx(1 - b)

                @pl.when(i >= 1)
                def _():
                    wait_scatter(1 - b, 0)

                start_gather(1 - b)

            wait_gather(b)
            for par in (0, 1):
                dd = 2 * par + b

                @pl.when(j % 2 == par)
                def _():
                    wait_didx(dd)
                    start_scatter(b, dd)
                    if with_deg:
                        pltpu.sync_copy(ones_v, dacc.at[didx[dd]], add=True)

                    @pl.when(i + 2 < _NCHUNK)
                    def _():
                        load_idx(i + 2, b, (dd + 2) % 4)
        return carry

    lax.fori_loop(0, _NDBL, dbl, 0)
    wait_scatter(0, 0)
    wait_scatter(1, 1)

    @pl.when(wid < 4)
    def _():
        off = _XBASE + wid * _CH
        pltpu.sync_copy(src.at[pl.ds(off, _CH)], idx0)
        pltpu.async_copy(h.at[idx0], rows0, gsem0).wait()
        pltpu.sync_copy(dst.at[pl.ds(off, _CH)], didx0)
        pltpu.sync_copy(rows0, acc.at[didx0], add=True)
        if with_deg:
            pltpu.sync_copy(ones_v, dacc.at[didx0], add=True)

    plsc.subcore_barrier()

    pltpu.sync_copy(acc.at[pl.ds(s * _RT, _RT)],
                    part.at[pl.ds(c * _N + s * _RT, _RT)])

    @pl.when(s == 0)
    def _():
        pltpu.sync_copy(acc.at[pl.ds(_REM0, _REM)],
                        part.at[pl.ds(c * _N + _REM0, _REM)])
    if with_deg:
        @pl.when(s == 0)
        def _():
            pltpu.sync_copy(dacc, degv)
            pltpu.sync_copy(degv, degpart.at[pl.ds(c * _N, _N)])


def _make_sc(with_deg):
    mesh = plsc.VectorSubcoreMesh(core_axis_name="c", subcore_axis_name="s")
    out_type = [jax.ShapeDtypeStruct((2 * _N, _H), _f32)]
    scratch = [
        pltpu.VMEM_SHARED((_N, _H), _f32),
    ]
    if with_deg:
        out_type.append(jax.ShapeDtypeStruct((2 * _N,), _f32))
        scratch.append(pltpu.VMEM_SHARED((_N,), _f32))
    scratch += [
        pltpu.VMEM((_CH,), jnp.int32),
        pltpu.VMEM((_CH,), jnp.int32),
        pltpu.VMEM((_CH,), jnp.int32),
        pltpu.VMEM((_CH,), jnp.int32),
        pltpu.VMEM((_CH,), jnp.int32),
        pltpu.VMEM((_CH,), jnp.int32),
        pltpu.VMEM((_CH, _H), _f32),
        pltpu.VMEM((_CH, _H), _f32),
    ]
    if with_deg:
        scratch.append(pltpu.VMEM((_CH,), _f32))
        scratch.append(pltpu.VMEM((_N,), _f32))
    scratch += [pltpu.SemaphoreType.DMA] * 10
    return pl.kernel(
        functools.partial(_sc_body, with_deg),
        out_type=out_type,
        mesh=mesh,
        scratch_types=scratch,
    )


def _embed_body(h_ref, wt_ref, b_ref, o_ref):
    o_ref[...] = (
        jnp.dot(h_ref[...], wt_ref[...], preferred_element_type=_f32)
        + b_ref[...]
    )


def _self_body(h_ref, wl_ref, b_ref, s_ref):
    s_ref[...] = (
        jnp.dot(h_ref[...], wl_ref[...], preferred_element_type=_f32)
        + b_ref[...]
    )


def _combine_body(first, last, *refs):
    if first:
        (h_ref, s_ref, p0_ref, p1_ref, d0_ref, d1_ref,
         wet_ref, be_ref, wr_ref, o_ref, inv_ref) = refs
        dsum = d0_ref[...] + d1_ref[...]
        inv = 1.0 / jnp.maximum(dsum, 1.0)
        mask = jnp.where(dsum > 0.0, 1.0, 0.0)
        rawagg = (p0_ref[...] + p1_ref[...]) * inv
        agg = (
            jnp.dot(rawagg, wet_ref[...], preferred_element_type=_f32)
            + be_ref[...] * mask
        )
    elif last:
        (h_ref, s_ref, p0_ref, p1_ref, inv_in_ref, wr_ref,
         o_ref, cs_ref) = refs
        agg = (p0_ref[...] + p1_ref[...]) * inv_in_ref[...]
    else:
        (h_ref, s_ref, p0_ref, p1_ref, inv_in_ref, wr_ref, o_ref) = refs
        agg = (p0_ref[...] + p1_ref[...]) * inv_in_ref[...]
    bundle = s_ref[...] + jnp.dot(agg, wr_ref[...],
                                  preferred_element_type=_f32)
    nrm = jnp.maximum(
        jnp.sqrt(jnp.sum(bundle * bundle, axis=1, keepdims=True)), 1e-12)
    out = h_ref[...] + jnp.maximum(bundle / nrm, 0.0)
    o_ref[...] = out
    if first:
        inv_ref[...] = inv
    if last:
        i = pl.program_id(0)
        colsum = jnp.sum(out, axis=0, keepdims=True)

        @pl.when(i == 0)
        def _():
            cs_ref[...] = colsum

        @pl.when(i > 0)
        def _():
            cs_ref[...] += colsum


def _readout_body(cs_ref, pp_ref, pn_ref, fc_ref, o_ref):
    hg = cs_ref[...] * (1.0 / _N)
    dp = jnp.sum((hg - pp_ref[...]) ** 2, axis=1, keepdims=True)
    dn = jnp.sum((hg - pn_ref[...]) ** 2, axis=1, keepdims=True)
    d = jnp.concatenate([dp, dn], axis=0)
    ss = jnp.log((d + 1.0) / (d + 1e-12))
    y = jnp.sum(ss * fc_ref[...])
    o_ref[...] = jnp.reshape(jax.nn.sigmoid(y), (1, 1))


def _row_spec(off=0):
    return pl.BlockSpec((_BLK, _H), lambda i: (i + off, 0))


def _full_spec(shape):
    return pl.BlockSpec(shape, lambda i: tuple(0 for _ in shape))


def _embed_call(h, wt, b):
    return pl.pallas_call(
        _embed_body,
        grid=(_NBLK,),
        in_specs=[_row_spec(), _full_spec((_H, _H)), _full_spec((1, _H))],
        out_specs=_row_spec(),
        out_shape=jax.ShapeDtypeStruct((_N, _H), _f32),
    )(h, wt, b)


def _self_call(h, wl, b):
    return pl.pallas_call(
        _self_body,
        grid=(_NBLK,),
        in_specs=[_row_spec(), _full_spec((_H, _H)), _full_spec((1, _H))],
        out_specs=_row_spec(),
        out_shape=jax.ShapeDtypeStruct((_N, _H), _f32),
    )(h, wl, b)


def _combine_call(first, last, h, s, part, *rest):
    col_spec = pl.BlockSpec((_BLK, 1), lambda i: (i, 0))
    col2_spec1 = pl.BlockSpec((_BLK, 1), lambda i: (i + _NBLK, 0))
    in_specs = [
        _row_spec(),
        _row_spec(),
        _row_spec(),
        _row_spec(_NBLK),
    ]
    args = [h, s, part, part]
    if first:
        deg2, wet, be, wr = rest
        in_specs += [col_spec, col2_spec1,
                     _full_spec((_H, _H)), _full_spec((1, _H)),
                     _full_spec((_H, _H))]
        args += [deg2, deg2, wet, be, wr]
    else:
        inv, wr = rest
        in_specs += [col_spec, _full_spec((_H, _H))]
        args += [inv, wr]
    out_specs = [_row_spec()]
    out_shape = [jax.ShapeDtypeStruct((_N, _H), _f32)]
    if first:
        out_specs.append(pl.BlockSpec((_BLK, 1), lambda i: (i, 0)))
        out_shape.append(jax.ShapeDtypeStruct((_N, 1), _f32))
    if last:
        out_specs.append(pl.BlockSpec((1, _H), lambda i: (0, 0)))
        out_shape.append(jax.ShapeDtypeStruct((1, _H), _f32))
    return pl.pallas_call(
        functools.partial(_combine_body, first, last),
        grid=(_NBLK,),
        in_specs=in_specs,
        out_specs=out_specs,
        out_shape=out_shape,
    )(*args)


def _readout_call(cs, pp, pn, fc):
    return pl.pallas_call(
        _readout_body,
        out_shape=jax.ShapeDtypeStruct((1, 1), _f32),
    )(cs, pp, pn, fc)


_sc_seg_deg = _make_sc(True)
_sc_seg = _make_sc(False)


def kernel(h, edge_index, e, W_embed, b_embed, W0, b0, W1, b1, W2, b2,
           p_pos, p_neg, FC_w):
    src = edge_index[0]
    dst = edge_index[1]
    zrows = jnp.zeros((_N, _H), _f32)
    zdeg = jnp.zeros((_N,), _f32)

    part1, degpart = _sc_seg_deg(h, src, dst, zrows, zdeg)
    h0 = _embed_call(h, W_embed.T, b_embed.reshape(1, _H))
    s1 = _self_call(h0, W0[:, :_H].T, b0.reshape(1, _H))
    h1, inv = _combine_call(True, False, h0, s1, part1,
                            degpart.reshape(2 * _N, 1),
                            W_embed.T, b_embed.reshape(1, _H),
                            W0[:, _H:].T)

    part2, = _sc_seg(h1, src, dst, zrows)
    s2 = _self_call(h1, W1[:, :_H].T, b1.reshape(1, _H))
    h2, = _combine_call(False, False, h1, s2, part2, inv, W1[:, _H:].T)

    part3, = _sc_seg(h2, src, dst, zrows)
    s3 = _self_call(h2, W2[:, :_H].T, b2.reshape(1, _H))
    _h3, cs = _combine_call(False, True, h2, s3, part3, inv, W2[:, _H:].T)

    y = _readout_call(cs, p_pos, p_neg, FC_w.reshape(8, 1))
    return y.reshape(())


# speedup vs baseline: 11.6658x; 1.0064x over previous
import functools

import jax
import jax.numpy as jnp
from jax import lax
from jax.experimental import pallas as pl
from jax.experimental.pallas import tpu as pltpu
from jax.experimental.pallas import tpu_sc as plsc

_N = 10000
_E = 320000
_H = 128
_NC = 2
_NS = 16
_NW = _NC * _NS
_EW = _E // _NW
_CH = 128
_EWM = 9984
_NCHUNK = _EWM // _CH
_NDBL = _NCHUNK // 2
_XBASE = _NW * _EWM
_RT = 624
_REM0 = _NS * _RT
_REM = _N - _REM0

_BLK = 1000
_NBLK = _N // _BLK

_f32 = jnp.float32


def _sc_body(with_deg, *refs):
    if with_deg:
        (h, src, dst, zrows, zdeg, part, degpart,
         acc, dacc, idx0, idx1, didx0, didx1, didx2, didx3, rows0, rows1,
         ones_v, degv,
         isem0, isem1, dsem0, dsem1, dsem2, dsem3,
         gsem0, gsem1, ssem0, ssem1) = refs
    else:
        (h, src, dst, zrows, part,
         acc, idx0, idx1, didx0, didx1, didx2, didx3, rows0, rows1,
         isem0, isem1, dsem0, dsem1, dsem2, dsem3,
         gsem0, gsem1, ssem0, ssem1) = refs
    idx = (idx0, idx1)
    didx = (didx0, didx1, didx2, didx3)
    rows = (rows0, rows1)
    isem = (isem0, isem1)
    dsem = (dsem0, dsem1, dsem2, dsem3)
    gsem = (gsem0, gsem1)
    ssem = (ssem0, ssem1)

    c = lax.axis_index("c")
    s = lax.axis_index("s")
    wid = s * _NC + c

    pltpu.sync_copy(zrows.at[pl.ds(s * _RT, _RT)], acc.at[pl.ds(s * _RT, _RT)])

    @pl.when(s == 0)
    def _():
        pltpu.sync_copy(zrows.at[pl.ds(_REM0, _REM)],
                        acc.at[pl.ds(_REM0, _REM)])
    if with_deg:
        @pl.when(s == 0)
        def _():
            pltpu.sync_copy(zdeg, degv)
            pltpu.sync_copy(degv, dacc)
        for k in range(_CH // 16):
            ones_v[pl.ds(k * 16, 16)] = jnp.ones((16,), _f32)
    plsc.subcore_barrier()

    base = wid * _EWM

    def load_idx(i, b, d):
        off = base + i * _CH
        pltpu.async_copy(src.at[pl.ds(off, _CH)], idx[b], isem[b])
        pltpu.async_copy(dst.at[pl.ds(off, _CH)], didx[d], dsem[d])

    def wait_idx(b):
        pltpu.make_async_copy(src.at[pl.ds(0, _CH)], idx[b], isem[b]).wait()

    def wait_didx(d):
        pltpu.make_async_copy(dst.at[pl.ds(0, _CH)], didx[d], dsem[d]).wait()

    def start_gather(b):
        pltpu.async_copy(h.at[idx[b]], rows[b], gsem[b])

    def wait_gather(b):
        pltpu.make_async_copy(h.at[idx[b]], rows[b], gsem[b]).wait()

    def start_scatter(b, d):
        pltpu.async_copy(rows[b], acc.at[didx[d]], ssem[b], add=True)

    def wait_scatter(b, d):
        pltpu.make_async_copy(rows[b], acc.at[didx[d]], ssem[b]).wait()

    load_idx(0, 0, 0)
    load_idx(1, 1, 1)
    wait_idx(0)
    start_gather(0)

    def dbl(j, carry):
        for b in (0, 1):
            i = 2 * j + b

            @pl.when(i + 1 < _NCHUNK)
            def _():
                wait_idx(1 - b)

                @pl.when(i >= 1)
                def _():
                    wait_scatter(1 - b, 0)

                start_gather(1 - b)

            wait_gather(b)
            for par in (0, 1):
                dd = 2 * par + b

                @pl.when(j % 2 == par)
                def _():
                    wait_didx(dd)
                    start_scatter(b, dd)
                    if with_deg:
                        pltpu.sync_copy(ones_v, dacc.at[didx[dd]], add=True)

                    @pl.when(i + 2 < _NCHUNK)
                    def _():
                        load_idx(i + 2, b, (dd + 2) % 4)
        return carry

    lax.fori_loop(0, _NDBL, dbl, 0)
    wait_scatter(0, 0)
    wait_scatter(1, 1)

    @pl.when(wid < 4)
    def _():
        off = _XBASE + wid * _CH
        pltpu.sync_copy(src.at[pl.ds(off, _CH)], idx0)
        pltpu.async_copy(h.at[idx0], rows0, gsem0).wait()
        pltpu.sync_copy(dst.at[pl.ds(off, _CH)], didx0)
        pltpu.sync_copy(rows0, acc.at[didx0], add=True)
        if with_deg:
            pltpu.sync_copy(ones_v, dacc.at[didx0], add=True)

    plsc.subcore_barrier()

    pltpu.sync_copy(acc.at[pl.ds(s * _RT, _RT)],
                    part.at[pl.ds(c * _N + s * _RT, _RT)])

    @pl.when(s == 0)
    def _():
        pltpu.sync_copy(acc.at[pl.ds(_REM0, _REM)],
                        part.at[pl.ds(c * _N + _REM0, _REM)])
    if with_deg:
        @pl.when(s == 0)
        def _():
            pltpu.sync_copy(dacc, degv)
            pltpu.sync_copy(degv, degpart.at[pl.ds(c * _N, _N)])


def _make_sc(with_deg):
    mesh = plsc.VectorSubcoreMesh(core_axis_name="c", subcore_axis_name="s")
    out_type = [jax.ShapeDtypeStruct((2 * _N, _H), _f32)]
    scratch = [
        pltpu.VMEM_SHARED((_N, _H), _f32),
    ]
    if with_deg:
        out_type.append(jax.ShapeDtypeStruct((2 * _N,), _f32))
        scratch.append(pltpu.VMEM_SHARED((_N,), _f32))
    scratch += [
        pltpu.VMEM((_CH,), jnp.int32),
        pltpu.VMEM((_CH,), jnp.int32),
        pltpu.VMEM((_CH,), jnp.int32),
        pltpu.VMEM((_CH,), jnp.int32),
        pltpu.VMEM((_CH,), jnp.int32),
        pltpu.VMEM((_CH,), jnp.int32),
        pltpu.VMEM((_CH, _H), _f32),
        pltpu.VMEM((_CH, _H), _f32),
    ]
    if with_deg:
        scratch.append(pltpu.VMEM((_CH,), _f32))
        scratch.append(pltpu.VMEM((_N,), _f32))
    scratch += [pltpu.SemaphoreType.DMA] * 10
    return pl.kernel(
        functools.partial(_sc_body, with_deg),
        out_type=out_type,
        mesh=mesh,
        scratch_types=scratch,
    )


def _embed_self_body(h_ref, wet_ref, be_ref, wl_ref, b_ref, h0_ref, s_ref):
    h0 = (
        jnp.dot(h_ref[...], wet_ref[...], preferred_element_type=_f32)
        + be_ref[...]
    )
    h0_ref[...] = h0
    s_ref[...] = (
        jnp.dot(h0, wl_ref[...], preferred_element_type=_f32) + b_ref[...]
    )


def _self_body(h_ref, wl_ref, b_ref, s_ref):
    s_ref[...] = (
        jnp.dot(h_ref[...], wl_ref[...], preferred_element_type=_f32)
        + b_ref[...]
    )


def _combine_body(first, last, *refs):
    if first:
        (h_ref, s_ref, p0_ref, p1_ref, d0_ref, d1_ref,
         wet_ref, be_ref, wr_ref, o_ref, inv_ref) = refs
        dsum = d0_ref[...] + d1_ref[...]
        inv = 1.0 / jnp.maximum(dsum, 1.0)
        mask = jnp.where(dsum > 0.0, 1.0, 0.0)
        rawagg = (p0_ref[...] + p1_ref[...]) * inv
        agg = (
            jnp.dot(rawagg, wet_ref[...], preferred_element_type=_f32)
            + be_ref[...] * mask
        )
    elif last:
        (h_ref, s_ref, p0_ref, p1_ref, inv_in_ref, wr_ref,
         pp_ref, pn_ref, fc_ref, o_ref, cs_ref, y_ref) = refs
        agg = (p0_ref[...] + p1_ref[...]) * inv_in_ref[...]
    else:
        (h_ref, s_ref, p0_ref, p1_ref, inv_in_ref, wr_ref, o_ref) = refs
        agg = (p0_ref[...] + p1_ref[...]) * inv_in_ref[...]
    bundle = s_ref[...] + jnp.dot(agg, wr_ref[...],
                                  preferred_element_type=_f32)
    nrm = jnp.maximum(
        jnp.sqrt(jnp.sum(bundle * bundle, axis=1, keepdims=True)), 1e-12)
    out = h_ref[...] + jnp.maximum(bundle / nrm, 0.0)
    o_ref[...] = out
    if first:
        inv_ref[...] = inv
    if last:
        i = pl.program_id(0)
        colsum = jnp.sum(out, axis=0, keepdims=True)

        @pl.when(i == 0)
        def _():
            cs_ref[...] = colsum

        @pl.when(i > 0)
        def _():
            cs_ref[...] += colsum

        @pl.when(i == _NBLK - 1)
        def _():
            hg = cs_ref[...] * (1.0 / _N)
            dp = jnp.sum((hg - pp_ref[...]) ** 2, axis=1, keepdims=True)
            dn = jnp.sum((hg - pn_ref[...]) ** 2, axis=1, keepdims=True)
            d = jnp.concatenate([dp, dn], axis=0)
            ss = jnp.log((d + 1.0) / (d + 1e-12))
            yv = jnp.sum(ss * fc_ref[...])
            y_ref[...] = jnp.reshape(jax.nn.sigmoid(yv), (1, 1))


def _row_spec(off=0):
    return pl.BlockSpec((_BLK, _H), lambda i: (i + off, 0))


def _full_spec(shape):
    return pl.BlockSpec(shape, lambda i: tuple(0 for _ in shape))


def _embed_self_call(h, wet, be, wl, b):
    return pl.pallas_call(
        _embed_self_body,
        grid=(_NBLK,),
        in_specs=[_row_spec(), _full_spec((_H, _H)), _full_spec((1, _H)),
                  _full_spec((_H, _H)), _full_spec((1, _H))],
        out_specs=[_row_spec(), _row_spec()],
        out_shape=[jax.ShapeDtypeStruct((_N, _H), _f32),
                   jax.ShapeDtypeStruct((_N, _H), _f32)],
    )(h, wet, be, wl, b)


def _self_call(h, wl, b):
    return pl.pallas_call(
        _self_body,
        grid=(_NBLK,),
        in_specs=[_row_spec(), _full_spec((_H, _H)), _full_spec((1, _H))],
        out_specs=_row_spec(),
        out_shape=jax.ShapeDtypeStruct((_N, _H), _f32),
    )(h, wl, b)


def _combine_call(first, last, h, s, part, *rest):
    col_spec = pl.BlockSpec((_BLK, 1), lambda i: (i, 0))
    col2_spec1 = pl.BlockSpec((_BLK, 1), lambda i: (i + _NBLK, 0))
    in_specs = [
        _row_spec(),
        _row_spec(),
        _row_spec(),
        _row_spec(_NBLK),
    ]
    args = [h, s, part, part]
    if first:
        deg2, wet, be, wr = rest
        in_specs += [col_spec, col2_spec1,
                     _full_spec((_H, _H)), _full_spec((1, _H)),
                     _full_spec((_H, _H))]
        args += [deg2, deg2, wet, be, wr]
    elif last:
        inv, wr, pp, pn, fc = rest
        in_specs += [col_spec, _full_spec((_H, _H)),
                     _full_spec((4, _H)), _full_spec((4, _H)),
                     _full_spec((8, 1))]
        args += [inv, wr, pp, pn, fc]
    else:
        inv, wr = rest
        in_specs += [col_spec, _full_spec((_H, _H))]
        args += [inv, wr]
    out_specs = [_row_spec()]
    out_shape = [jax.ShapeDtypeStruct((_N, _H), _f32)]
    if first:
        out_specs.append(pl.BlockSpec((_BLK, 1), lambda i: (i, 0)))
        out_shape.append(jax.ShapeDtypeStruct((_N, 1), _f32))
    if last:
        out_specs.append(pl.BlockSpec((1, _H), lambda i: (0, 0)))
        out_shape.append(jax.ShapeDtypeStruct((1, _H), _f32))
        out_specs.append(pl.BlockSpec((1, 1), lambda i: (0, 0)))
        out_shape.append(jax.ShapeDtypeStruct((1, 1), _f32))
    return pl.pallas_call(
        functools.partial(_combine_body, first, last),
        grid=(_NBLK,),
        in_specs=in_specs,
        out_specs=out_specs,
        out_shape=out_shape,
    )(*args)


_sc_seg_deg = _make_sc(True)
_sc_seg = _make_sc(False)


def kernel(h, edge_index, e, W_embed, b_embed, W0, b0, W1, b1, W2, b2,
           p_pos, p_neg, FC_w):
    src = edge_index[0]
    dst = edge_index[1]
    zrows = jnp.zeros((_N, _H), _f32)
    zdeg = jnp.zeros((_N,), _f32)

    part1, degpart = _sc_seg_deg(h, src, dst, zrows, zdeg)
    h0, s1 = _embed_self_call(h, W_embed.T, b_embed.reshape(1, _H),
                              W0[:, :_H].T, b0.reshape(1, _H))
    h1, inv = _combine_call(True, False, h0, s1, part1,
                            degpart.reshape(2 * _N, 1),
                            W_embed.T, b_embed.reshape(1, _H),
                            W0[:, _H:].T)

    part2, = _sc_seg(h1, src, dst, zrows)
    s2 = _self_call(h1, W1[:, :_H].T, b1.reshape(1, _H))
    h2, = _combine_call(False, False, h1, s2, part2, inv, W1[:, _H:].T)

    part3, = _sc_seg(h2, src, dst, zrows)
    s3 = _self_call(h2, W2[:, :_H].T, b2.reshape(1, _H))
    _h3, _cs, y = _combine_call(False, True, h2, s3, part3, inv,
                                W2[:, _H:].T, p_pos, p_neg,
                                FC_w.reshape(8, 1))
    return y.reshape(())
